# Initial kernel scaffold; baseline (speedup 1.0000x reference)
import jax
import jax.numpy as jnp
from jax.experimental import pallas as pl


def kernel(x, edge_index, eps, W1, b1, W2, b2):
    raise NotImplementedError("write your pallas kernel here")


# speedup vs baseline: 8.8397x; 8.8397x over previous
import functools

import jax
import jax.numpy as jnp
from jax import lax
from jax.experimental import pallas as pl
from jax.experimental.pallas import tpu as pltpu
from jax.experimental.pallas import tpu_sc as plsc

N = 10000
E = 320000
D = 128
DH = D // 2

NC = 2
NS = 16

E_PER_S = E // NS
CHUNK = 80
NCH = E_PER_S // CHUNK
WB_TILES = 10
WB_ROWS = N // WB_TILES
ZROWS = 200


def _sc_aggregate(x_split, src_idx, dst_idx):
    mesh = plsc.VectorSubcoreMesh(core_axis_name="c", subcore_axis_name="s")

    @functools.partial(
        pl.kernel,
        mesh=mesh,
        out_type=jax.ShapeDtypeStruct((NC, N, DH), jnp.float32),
        scratch_types=[
            pltpu.VMEM((NCH, CHUNK), jnp.int32),
            pltpu.VMEM((NCH, CHUNK), jnp.int32),
            pltpu.VMEM((CHUNK, DH), jnp.float32),
            pltpu.VMEM((CHUNK, DH), jnp.float32),
            pltpu.VMEM((ZROWS, DH), jnp.float32),
            pltpu.VMEM_SHARED((N, DH), jnp.float32),
            pltpu.SemaphoreType.DMA,
            pltpu.SemaphoreType.DMA,
        ],
        compiler_params=pltpu.CompilerParams(use_tc_tiling_on_sc=False),
    )
    def agg_kernel(x_hbm, src_hbm, dst_hbm, out_hbm,
                   sidx, didx, rows0, rows1, zbuf, acc, sem0, sem1):
        cid = lax.axis_index("c")
        sid = lax.axis_index("s")

        pltpu.sync_copy(src_hbm.at[sid], sidx)
        pltpu.sync_copy(dst_hbm.at[sid], didx)

        def zrow(i, _):
            for j in range(DH // 16):
                zbuf[i, pl.ds(j * 16, 16)] = jnp.zeros((16,), jnp.float32)
            return 0
        lax.fori_loop(0, ZROWS, zrow, 0)
        base = sid * WB_ROWS

        @pl.when(sid < WB_TILES)
        def _():
            for k in range(WB_ROWS // ZROWS):
                pltpu.sync_copy(zbuf, acc.at[pl.ds(base + k * ZROWS, ZROWS)])
        plsc.subcore_barrier()

        xh = x_hbm.at[cid]

        pltpu.async_copy(xh.at[sidx.at[0]], rows0, sem0)

        def step(c, _):
            @pl.when(c % 2 == 1)
            def _():
                pltpu.async_copy(xh.at[sidx.at[c]], rows1, sem1)
                pltpu.make_async_copy(xh.at[pl.ds(0, CHUNK)], rows0, sem0).wait()
                pltpu.sync_copy(rows0, acc.at[didx.at[c - 1]], add=True)

            @pl.when(c % 2 == 0)
            def _():
                pltpu.async_copy(xh.at[sidx.at[c]], rows0, sem0)
                pltpu.make_async_copy(xh.at[pl.ds(0, CHUNK)], rows1, sem1).wait()
                pltpu.sync_copy(rows1, acc.at[didx.at[c - 1]], add=True)
            return 0

        lax.fori_loop(1, NCH, step, 0)
        pltpu.make_async_copy(xh.at[pl.ds(0, CHUNK)], rows1, sem1).wait()
        pltpu.sync_copy(rows1, acc.at[didx.at[NCH - 1]], add=True)

        plsc.subcore_barrier()

        @pl.when(sid < WB_TILES)
        def _():
            pltpu.sync_copy(acc.at[pl.ds(base, WB_ROWS)],
                            out_hbm.at[cid, pl.ds(base, WB_ROWS)])

    return agg_kernel(x_split, src_idx, dst_idx)


def _tc_mlp_block(scale_ref, x_ref, p0_ref, p1_ref, w1_ref, b1_ref,
                  w2_ref, b2_ref, out_ref):
    agg = jnp.concatenate([p0_ref[0], p1_ref[0]], axis=1)
    t = scale_ref[0, 0] * x_ref[...] + agg
    h = jnp.dot(t, w1_ref[...], preferred_element_type=jnp.float32) + b1_ref[...]
    h = jnp.maximum(h, 0.0)
    out_ref[...] = (
        jnp.dot(h, w2_ref[...], preferred_element_type=jnp.float32) + b2_ref[...]
    )


def _tc_mlp(x, partials, scale, W1, b1, W2, b2):
    rows = 1000
    grid = (N // rows,)
    return pl.pallas_call(
        _tc_mlp_block,
        grid=grid,
        in_specs=[
            pl.BlockSpec(memory_space=pltpu.SMEM),
            pl.BlockSpec((rows, D), lambda i: (i, 0)),
            pl.BlockSpec((1, rows, DH), lambda i: (0, i, 0)),
            pl.BlockSpec((1, rows, DH), lambda i: (1, i, 0)),
            pl.BlockSpec((D, D), lambda i: (0, 0)),
            pl.BlockSpec((1, D), lambda i: (0, 0)),
            pl.BlockSpec((D, D), lambda i: (0, 0)),
            pl.BlockSpec((1, D), lambda i: (0, 0)),
        ],
        out_specs=pl.BlockSpec((rows, D), lambda i: (i, 0)),
        out_shape=jax.ShapeDtypeStruct((N, D), jnp.float32),
    )(scale, x, partials, partials, W1, b1, W2, b2)


def kernel(x, edge_index, eps, W1, b1, W2, b2):
    ei = edge_index.astype(jnp.int32)
    src = ei[0].reshape(NS, NCH, CHUNK)
    dst = ei[1].reshape(NS, NCH, CHUNK)
    x_split = x.reshape(N, NC, DH).transpose(1, 0, 2)
    partials = _sc_aggregate(x_split, src, dst)
    scale = (1.0 + eps).astype(jnp.float32).reshape(1, 1)
    return _tc_mlp(x, partials, scale, W1.astype(jnp.float32),
                   b1.reshape(1, D), W2.astype(jnp.float32), b2.reshape(1, D))


# speedup vs baseline: 11.4050x; 1.2902x over previous
import functools

import jax
import jax.numpy as jnp
from jax import lax
from jax.experimental import pallas as pl
from jax.experimental.pallas import tpu as pltpu
from jax.experimental.pallas import tpu_sc as plsc

N = 10000
E = 320000
D = 128
DH = D // 2

NC = 2
NS = 16

E_PER_S = E // NS
CHUNK = 80
NCH = E_PER_S // CHUNK
NB = 5
WB_TILES = 10
WB_ROWS = N // WB_TILES
ZROWS = 200


def _sc_aggregate(x_split, src_idx, dst_idx):
    mesh = plsc.VectorSubcoreMesh(core_axis_name="c", subcore_axis_name="s")

    @functools.partial(
        pl.kernel,
        mesh=mesh,
        out_type=jax.ShapeDtypeStruct((NC, N, DH), jnp.float32),
        scratch_types=[
            pltpu.VMEM((NCH, CHUNK), jnp.int32),
            pltpu.VMEM((NCH, CHUNK), jnp.int32),
            pltpu.VMEM((NB, CHUNK, DH), jnp.float32),
            pltpu.VMEM((ZROWS, DH), jnp.float32),
            pltpu.VMEM_SHARED((N, DH), jnp.float32),
            [pltpu.SemaphoreType.DMA] * NB,
            [pltpu.SemaphoreType.DMA] * NB,
        ],
        compiler_params=pltpu.CompilerParams(use_tc_tiling_on_sc=False),
    )
    def agg_kernel(x_hbm, src_hbm, dst_hbm, out_hbm,
                   sidx, didx, rows, zbuf, acc, gsem, ssem):
        cid = lax.axis_index("c")
        sid = lax.axis_index("s")

        pltpu.sync_copy(src_hbm.at[sid], sidx)
        pltpu.sync_copy(dst_hbm.at[sid], didx)

        xh = x_hbm.at[cid]

        def wait_gather(b):
            pltpu.make_async_copy(xh.at[pl.ds(0, CHUNK)], rows.at[b], gsem[b]).wait()

        def wait_scatter(b):
            pltpu.make_async_copy(xh.at[pl.ds(0, CHUNK)], rows.at[b], ssem[b]).wait()

        pltpu.async_copy(xh.at[sidx.at[0]], rows.at[0], gsem[0])
        pltpu.async_copy(xh.at[sidx.at[1]], rows.at[1], gsem[1])

        def zrow(i, _):
            for j in range(DH // 16):
                zbuf[i, pl.ds(j * 16, 16)] = jnp.zeros((16,), jnp.float32)
            return 0
        lax.fori_loop(0, ZROWS, zrow, 0)
        base = sid * WB_ROWS

        @pl.when(sid < WB_TILES)
        def _():
            for k in range(WB_ROWS // ZROWS):
                pltpu.sync_copy(zbuf, acc.at[pl.ds(base + k * ZROWS, ZROWS)])
        plsc.subcore_barrier()

        def group(g, _):
            for b in range(NB):
                c = g * NB + b
                b2 = (b + 2) % NB
                c2 = c + 2

                @pl.when(c2 < NCH)
                def _():
                    @pl.when(c >= NB - 2)
                    def _():
                        wait_scatter(b2)
                    pltpu.async_copy(xh.at[sidx.at[c2]], rows.at[b2], gsem[b2])

                wait_gather(b)
                pltpu.async_copy(rows.at[b], acc.at[didx.at[c]], ssem[b],
                                 add=True)
            return 0

        lax.fori_loop(0, NCH // NB, group, 0)
        for b in range(NB):
            wait_scatter(b)

        plsc.subcore_barrier()

        @pl.when(sid < WB_TILES)
        def _():
            pltpu.sync_copy(acc.at[pl.ds(base, WB_ROWS)],
                            out_hbm.at[cid, pl.ds(base, WB_ROWS)])

    return agg_kernel(x_split, src_idx, dst_idx)


def _tc_mlp_block(scale_ref, x_ref, p0_ref, p1_ref, w1_ref, b1_ref,
                  w2_ref, b2_ref, out_ref):
    agg = jnp.concatenate([p0_ref[0], p1_ref[0]], axis=1)
    t = scale_ref[0, 0] * x_ref[...] + agg
    h = jnp.dot(t, w1_ref[...], preferred_element_type=jnp.float32) + b1_ref[...]
    h = jnp.maximum(h, 0.0)
    out_ref[...] = (
        jnp.dot(h, w2_ref[...], preferred_element_type=jnp.float32) + b2_ref[...]
    )


def _tc_mlp(x, partials, scale, W1, b1, W2, b2):
    rows = 1000
    grid = (N // rows,)
    return pl.pallas_call(
        _tc_mlp_block,
        grid=grid,
        in_specs=[
            pl.BlockSpec(memory_space=pltpu.SMEM),
            pl.BlockSpec((rows, D), lambda i: (i, 0)),
            pl.BlockSpec((1, rows, DH), lambda i: (0, i, 0)),
            pl.BlockSpec((1, rows, DH), lambda i: (1, i, 0)),
            pl.BlockSpec((D, D), lambda i: (0, 0)),
            pl.BlockSpec((1, D), lambda i: (0, 0)),
            pl.BlockSpec((D, D), lambda i: (0, 0)),
            pl.BlockSpec((1, D), lambda i: (0, 0)),
        ],
        out_specs=pl.BlockSpec((rows, D), lambda i: (i, 0)),
        out_shape=jax.ShapeDtypeStruct((N, D), jnp.float32),
    )(scale, x, partials, partials, W1, b1, W2, b2)


def kernel(x, edge_index, eps, W1, b1, W2, b2):
    ei = edge_index.astype(jnp.int32)
    src = ei[0].reshape(NS, NCH, CHUNK)
    dst = ei[1].reshape(NS, NCH, CHUNK)
    x_split = x.reshape(N, NC, DH).transpose(1, 0, 2)
    partials = _sc_aggregate(x_split, src, dst)
    scale = (1.0 + eps).astype(jnp.float32).reshape(1, 1)
    return _tc_mlp(x, partials, scale, W1.astype(jnp.float32),
                   b1.reshape(1, D), W2.astype(jnp.float32), b2.reshape(1, D))


# speedup vs baseline: 12.5645x; 1.1017x over previous
import functools

import jax
import jax.numpy as jnp
from jax import lax
from jax.experimental import pallas as pl
from jax.experimental.pallas import tpu as pltpu
from jax.experimental.pallas import tpu_sc as plsc

N = 10000
E = 320000
D = 128
DH = D // 2

NC = 2
NS = 16

E_PER_S = E // NS
CHUNK = 80
NCH = E_PER_S // CHUNK
NB = 5
WB_TILES = 10
WB_ROWS = N // WB_TILES
ZROWS = 200


def _sc_aggregate(x2, src_idx, dst_idx):
    mesh = plsc.VectorSubcoreMesh(core_axis_name="c", subcore_axis_name="s")

    @functools.partial(
        pl.kernel,
        mesh=mesh,
        out_type=jax.ShapeDtypeStruct((N, D), jnp.float32),
        scratch_types=[
            pltpu.VMEM((NCH, CHUNK), jnp.int32),
            pltpu.VMEM((NCH, CHUNK), jnp.int32),
            pltpu.VMEM((NB, CHUNK, DH), jnp.float32),
            pltpu.VMEM((ZROWS, DH), jnp.float32),
            pltpu.VMEM_SHARED((N, DH), jnp.float32),
            [pltpu.SemaphoreType.DMA] * NB,
            [pltpu.SemaphoreType.DMA] * NB,
        ],
        compiler_params=pltpu.CompilerParams(use_tc_tiling_on_sc=False),
    )
    def agg_kernel(x_hbm, src_hbm, dst_hbm, out_hbm,
                   sidx, didx, rows, zbuf, acc, gsem, ssem):
        cid = lax.axis_index("c")
        sid = lax.axis_index("s")

        pltpu.sync_copy(src_hbm.at[cid, sid], sidx)
        pltpu.sync_copy(dst_hbm.at[sid], didx)

        def wait_gather(b):
            pltpu.make_async_copy(x_hbm.at[pl.ds(0, CHUNK)], rows.at[b], gsem[b]).wait()

        def wait_scatter(b):
            pltpu.make_async_copy(x_hbm.at[pl.ds(0, CHUNK)], rows.at[b], ssem[b]).wait()

        pltpu.async_copy(x_hbm.at[sidx.at[0]], rows.at[0], gsem[0])
        pltpu.async_copy(x_hbm.at[sidx.at[1]], rows.at[1], gsem[1])

        def zrow(i, _):
            for j in range(DH // 16):
                zbuf[i, pl.ds(j * 16, 16)] = jnp.zeros((16,), jnp.float32)
            return 0
        lax.fori_loop(0, ZROWS, zrow, 0)
        base = sid * WB_ROWS

        @pl.when(sid < WB_TILES)
        def _():
            for k in range(WB_ROWS // ZROWS):
                pltpu.sync_copy(zbuf, acc.at[pl.ds(base + k * ZROWS, ZROWS)])
        plsc.subcore_barrier()

        def group(g, _):
            for b in range(NB):
                c = g * NB + b
                b2 = (b + 2) % NB
                c2 = c + 2

                @pl.when(c2 < NCH)
                def _():
                    @pl.when(c >= NB - 2)
                    def _():
                        wait_scatter(b2)
                    pltpu.async_copy(x_hbm.at[sidx.at[c2]], rows.at[b2], gsem[b2])

                wait_gather(b)
                pltpu.async_copy(rows.at[b], acc.at[didx.at[c]], ssem[b],
                                 add=True)
            return 0

        lax.fori_loop(0, NCH // NB, group, 0)
        for b in range(NB):
            wait_scatter(b)

        plsc.subcore_barrier()

        @pl.when(sid < WB_TILES)
        def _():
            pltpu.sync_copy(acc.at[pl.ds(base, WB_ROWS)],
                            out_hbm.at[pl.ds(base, WB_ROWS),
                                       pl.ds(cid * DH, DH)])

    return agg_kernel(x2, src_idx, dst_idx)


def _tc_mlp_block(scale_ref, x_ref, p_ref, w1_ref, b1_ref,
                  w2_ref, b2_ref, out_ref):
    t = scale_ref[0, 0] * x_ref[...] + p_ref[...]
    h = jnp.dot(t, w1_ref[...], preferred_element_type=jnp.float32) + b1_ref[...]
    h = jnp.maximum(h, 0.0)
    out_ref[...] = (
        jnp.dot(h, w2_ref[...], preferred_element_type=jnp.float32) + b2_ref[...]
    )


def _tc_mlp(x, agg, scale, W1, b1, W2, b2):
    rows = 1000
    grid = (N // rows,)
    return pl.pallas_call(
        _tc_mlp_block,
        grid=grid,
        in_specs=[
            pl.BlockSpec(memory_space=pltpu.SMEM),
            pl.BlockSpec((rows, D), lambda i: (i, 0)),
            pl.BlockSpec((rows, D), lambda i: (i, 0)),
            pl.BlockSpec((D, D), lambda i: (0, 0)),
            pl.BlockSpec((1, D), lambda i: (0, 0)),
            pl.BlockSpec((D, D), lambda i: (0, 0)),
            pl.BlockSpec((1, D), lambda i: (0, 0)),
        ],
        out_specs=pl.BlockSpec((rows, D), lambda i: (i, 0)),
        out_shape=jax.ShapeDtypeStruct((N, D), jnp.float32),
    )(scale, x, agg, W1, b1, W2, b2)


def kernel(x, edge_index, eps, W1, b1, W2, b2):
    ei = edge_index.astype(jnp.int32)
    src = ei[0].reshape(1, NS, NCH, CHUNK)
    src2 = jnp.concatenate([2 * src, 2 * src + 1], axis=0)
    dst = ei[1].reshape(NS, NCH, CHUNK)
    x2 = x.reshape(2 * N, DH)
    agg = _sc_aggregate(x2, src2, dst)
    scale = (1.0 + eps).astype(jnp.float32).reshape(1, 1)
    return _tc_mlp(x, agg, scale, W1.astype(jnp.float32),
                   b1.reshape(1, D), W2.astype(jnp.float32), b2.reshape(1, D))


# speedup vs baseline: 13.4453x; 1.0701x over previous
import functools

import jax
import jax.numpy as jnp
from jax import lax
from jax.experimental import pallas as pl
from jax.experimental.pallas import tpu as pltpu
from jax.experimental.pallas import tpu_sc as plsc

N = 10000
E = 320000
D = 128
DH = D // 2

NC = 2
NS = 16

E_PER_S = E // NS
CHUNK = 80
NCH = E_PER_S // CHUNK
NB = 5
WB_TILES = 10
WB_ROWS = N // WB_TILES
ZROWS = 200


def _sc_aggregate(x, src_idx, dst_idx):
    mesh = plsc.VectorSubcoreMesh(core_axis_name="c", subcore_axis_name="s")

    @functools.partial(
        pl.kernel,
        mesh=mesh,
        out_type=jax.ShapeDtypeStruct((N, D), jnp.float32),
        scratch_types=[
            pltpu.VMEM((NCH, CHUNK), jnp.int32),
            pltpu.VMEM((NCH, CHUNK), jnp.int32),
            pltpu.VMEM((NB, CHUNK, DH), jnp.float32),
            pltpu.VMEM((ZROWS, DH), jnp.float32),
            pltpu.VMEM_SHARED((N, DH), jnp.float32),
            [pltpu.SemaphoreType.DMA] * NB,
            [pltpu.SemaphoreType.DMA] * NB,
        ],
        compiler_params=pltpu.CompilerParams(use_tc_tiling_on_sc=False),
    )
    def agg_kernel(x_hbm, src_hbm, dst_hbm, out_hbm,
                   sidx, didx, rows, zbuf, acc, gsem, ssem):
        cid = lax.axis_index("c")
        sid = lax.axis_index("s")
        xv = x_hbm

        pltpu.sync_copy(src_hbm.at[sid], sidx)
        pltpu.sync_copy(dst_hbm.at[sid], didx)

        def remap(c):
            for j in range(CHUNK // 16):
                s = pl.ds(j * 16, 16)
                sidx[c, s] = 2 * sidx[c, s] + cid

        def wait_gather(b):
            pltpu.make_async_copy(xv.at[pl.ds(0, CHUNK)], rows.at[b], gsem[b]).wait()

        def wait_scatter(b):
            pltpu.make_async_copy(xv.at[pl.ds(0, CHUNK)], rows.at[b], ssem[b]).wait()

        remap(0)
        remap(1)
        pltpu.async_copy(xv.at[sidx.at[0]], rows.at[0], gsem[0])
        pltpu.async_copy(xv.at[sidx.at[1]], rows.at[1], gsem[1])

        def zrow(i, _):
            for j in range(DH // 16):
                zbuf[i, pl.ds(j * 16, 16)] = jnp.zeros((16,), jnp.float32)
            return 0
        lax.fori_loop(0, ZROWS, zrow, 0)
        base = sid * WB_ROWS

        @pl.when(sid < WB_TILES)
        def _():
            for k in range(WB_ROWS // ZROWS):
                pltpu.sync_copy(zbuf, acc.at[pl.ds(base + k * ZROWS, ZROWS)])
        plsc.subcore_barrier()

        def group(g, _):
            for b in range(NB):
                c = g * NB + b
                b2 = (b + 2) % NB
                c2 = c + 2

                @pl.when(c2 < NCH)
                def _():
                    @pl.when(c >= NB - 2)
                    def _():
                        wait_scatter(b2)
                    remap(c2)
                    pltpu.async_copy(xv.at[sidx.at[c2]], rows.at[b2], gsem[b2])

                wait_gather(b)
                pltpu.async_copy(rows.at[b], acc.at[didx.at[c]], ssem[b],
                                 add=True)
            return 0

        lax.fori_loop(0, NCH // NB, group, 0)
        for b in range(NB):
            wait_scatter(b)

        plsc.subcore_barrier()

        @pl.when(sid < WB_TILES)
        def _():
            pltpu.sync_copy(acc.at[pl.ds(base, WB_ROWS)],
                            out_hbm.at[pl.ds(base, WB_ROWS),
                                       pl.ds(cid * DH, DH)])

    return agg_kernel(x, src_idx, dst_idx)


def _tc_mlp_block(scale_ref, x_ref, p_ref, w1_ref, b1_ref,
                  w2_ref, b2_ref, out_ref):
    t = scale_ref[0, 0] * x_ref[...] + p_ref[...]
    h = jnp.dot(t, w1_ref[...], preferred_element_type=jnp.float32) + b1_ref[...]
    h = jnp.maximum(h, 0.0)
    out_ref[...] = (
        jnp.dot(h, w2_ref[...], preferred_element_type=jnp.float32) + b2_ref[...]
    )


def _tc_mlp(x, agg, scale, W1, b1, W2, b2):
    rows = 1000
    grid = (N // rows,)
    return pl.pallas_call(
        _tc_mlp_block,
        grid=grid,
        in_specs=[
            pl.BlockSpec(memory_space=pltpu.SMEM),
            pl.BlockSpec((rows, D), lambda i: (i, 0)),
            pl.BlockSpec((rows, D), lambda i: (i, 0)),
            pl.BlockSpec((D, D), lambda i: (0, 0)),
            pl.BlockSpec((1, D), lambda i: (0, 0)),
            pl.BlockSpec((D, D), lambda i: (0, 0)),
            pl.BlockSpec((1, D), lambda i: (0, 0)),
        ],
        out_specs=pl.BlockSpec((rows, D), lambda i: (i, 0)),
        out_shape=jax.ShapeDtypeStruct((N, D), jnp.float32),
    )(scale, x, agg, W1, b1, W2, b2)


def kernel(x, edge_index, eps, W1, b1, W2, b2):
    ei = edge_index.astype(jnp.int32)
    src = ei[0].reshape(NS, NCH, CHUNK)
    dst = ei[1].reshape(NS, NCH, CHUNK)
    agg = _sc_aggregate(x.reshape(2 * N, DH), src, dst)
    scale = (1.0 + eps).astype(jnp.float32).reshape(1, 1)
    return _tc_mlp(x, agg, scale, W1.astype(jnp.float32),
                   b1.reshape(1, D), W2.astype(jnp.float32), b2.reshape(1, D))


# speedup vs baseline: 13.4640x; 1.0014x over previous
import functools

import jax
import jax.numpy as jnp
from jax import lax
from jax.experimental import pallas as pl
from jax.experimental.pallas import tpu as pltpu
from jax.experimental.pallas import tpu_sc as plsc

N = 10000
E = 320000
D = 128
DH = D // 2

NC = 2
NS = 16

E_PER_S = E // NS
CHUNK = 80
NCH = E_PER_S // CHUNK
NB = 5
WB_TILES = 10
WB_ROWS = N // WB_TILES
ZROWS = 200


def _sc_aggregate(x, src_idx, dst_idx):
    mesh = plsc.VectorSubcoreMesh(core_axis_name="c", subcore_axis_name="s")

    @functools.partial(
        pl.kernel,
        mesh=mesh,
        out_type=jax.ShapeDtypeStruct((N, D), jnp.float32),
        scratch_types=[
            pltpu.VMEM((NCH, CHUNK), jnp.int32),
            pltpu.VMEM((NCH, CHUNK), jnp.int32),
            pltpu.VMEM((NB, CHUNK, DH), jnp.float32),
            pltpu.VMEM((ZROWS, DH), jnp.float32),
            pltpu.VMEM_SHARED((N, DH), jnp.float32),
            [pltpu.SemaphoreType.DMA] * NB,
            [pltpu.SemaphoreType.DMA] * NB,
        ],
        compiler_params=pltpu.CompilerParams(use_tc_tiling_on_sc=False),
    )
    def agg_kernel(x_hbm, src_hbm, dst_hbm, out_hbm,
                   sidx, didx, rows, zbuf, acc, gsem, ssem):
        cid = lax.axis_index("c")
        sid = lax.axis_index("s")
        xv = x_hbm

        pltpu.sync_copy(src_hbm.at[sid], sidx)
        pltpu.sync_copy(dst_hbm.at[sid], didx)

        def remap(c):
            for j in range(CHUNK // 16):
                s = pl.ds(j * 16, 16)
                sidx[c, s] = 2 * sidx[c, s] + cid

        def wait_gather(b):
            pltpu.make_async_copy(xv.at[pl.ds(0, CHUNK)], rows.at[b], gsem[b]).wait()

        def wait_scatter(b):
            pltpu.make_async_copy(xv.at[pl.ds(0, CHUNK)], rows.at[b], ssem[b]).wait()

        remap(0)
        remap(1)
        remap(2)
        pltpu.async_copy(xv.at[sidx.at[0]], rows.at[0], gsem[0])
        pltpu.async_copy(xv.at[sidx.at[1]], rows.at[1], gsem[1])

        def zrow(i, _):
            for j in range(DH // 16):
                zbuf[i, pl.ds(j * 16, 16)] = jnp.zeros((16,), jnp.float32)
            return 0
        lax.fori_loop(0, ZROWS, zrow, 0)
        base = sid * WB_ROWS

        @pl.when(sid < WB_TILES)
        def _():
            for k in range(WB_ROWS // ZROWS):
                pltpu.sync_copy(zbuf, acc.at[pl.ds(base + k * ZROWS, ZROWS)])
        plsc.subcore_barrier()

        def group(g, _):
            for b in range(NB):
                c = g * NB + b
                b2 = (b + 2) % NB
                c2 = c + 2

                @pl.when(c + 3 < NCH)
                def _():
                    remap(c + 3)

                @pl.when(c2 < NCH)
                def _():
                    @pl.when(c >= NB - 2)
                    def _():
                        wait_scatter(b2)
                    pltpu.async_copy(xv.at[sidx.at[c2]], rows.at[b2], gsem[b2])

                wait_gather(b)
                pltpu.async_copy(rows.at[b], acc.at[didx.at[c]], ssem[b],
                                 add=True)
            return 0

        lax.fori_loop(0, NCH // NB, group, 0)
        for b in range(NB):
            wait_scatter(b)

        plsc.subcore_barrier()

        @pl.when(sid < WB_TILES)
        def _():
            pltpu.sync_copy(acc.at[pl.ds(base, WB_ROWS)],
                            out_hbm.at[pl.ds(base, WB_ROWS),
                                       pl.ds(cid * DH, DH)])

    return agg_kernel(x, src_idx, dst_idx)


def _tc_mlp_block(scale_ref, x_ref, p_ref, w1_ref, b1_ref,
                  w2_ref, b2_ref, out_ref):
    t = scale_ref[0, 0] * x_ref[...] + p_ref[...]
    h = jnp.dot(t, w1_ref[...], preferred_element_type=jnp.float32) + b1_ref[...]
    h = jnp.maximum(h, 0.0)
    out_ref[...] = (
        jnp.dot(h, w2_ref[...], preferred_element_type=jnp.float32) + b2_ref[...]
    )


def _tc_mlp(x, agg, scale, W1, b1, W2, b2):
    rows = 1000
    grid = (N // rows,)
    return pl.pallas_call(
        _tc_mlp_block,
        grid=grid,
        in_specs=[
            pl.BlockSpec(memory_space=pltpu.SMEM),
            pl.BlockSpec((rows, D), lambda i: (i, 0)),
            pl.BlockSpec((rows, D), lambda i: (i, 0)),
            pl.BlockSpec((D, D), lambda i: (0, 0)),
            pl.BlockSpec((1, D), lambda i: (0, 0)),
            pl.BlockSpec((D, D), lambda i: (0, 0)),
            pl.BlockSpec((1, D), lambda i: (0, 0)),
        ],
        out_specs=pl.BlockSpec((rows, D), lambda i: (i, 0)),
        out_shape=jax.ShapeDtypeStruct((N, D), jnp.float32),
    )(scale, x, agg, W1, b1, W2, b2)


def kernel(x, edge_index, eps, W1, b1, W2, b2):
    ei = edge_index.astype(jnp.int32)
    src = ei[0].reshape(NS, NCH, CHUNK)
    dst = ei[1].reshape(NS, NCH, CHUNK)
    agg = _sc_aggregate(x.reshape(2 * N, DH), src, dst)
    scale = (1.0 + eps).astype(jnp.float32).reshape(1, 1)
    return _tc_mlp(x, agg, scale, W1.astype(jnp.float32),
                   b1.reshape(1, D), W2.astype(jnp.float32), b2.reshape(1, D))


# speedup vs baseline: 15.2886x; 1.1355x over previous
import functools

import jax
import jax.numpy as jnp
from jax import lax
from jax.experimental import pallas as pl
from jax.experimental.pallas import tpu as pltpu
from jax.experimental.pallas import tpu_sc as plsc

N = 10000
E = 320000
D = 128
DH = D // 2

NC = 2
NS = 16

CHUNK = 128
NBLK = E // CHUNK
BPT = NBLK // NS
XTRA = NBLK - BPT * NS
NB = 4
WB_TILES = 10
WB_ROWS = N // WB_TILES
ZROWS = 200


def _sc_aggregate(x2, eidx):
    mesh = plsc.VectorSubcoreMesh(core_axis_name="c", subcore_axis_name="s")

    @functools.partial(
        pl.kernel,
        mesh=mesh,
        out_type=jax.ShapeDtypeStruct((N, D), jnp.float32),
        scratch_types=[
            pltpu.VMEM((BPT + 1, 2, CHUNK), jnp.int32),
            pltpu.VMEM((NB, CHUNK, DH), jnp.float32),
            pltpu.VMEM((ZROWS, DH), jnp.float32),
            pltpu.VMEM_SHARED((N, DH), jnp.float32),
            [pltpu.SemaphoreType.DMA] * NB,
            [pltpu.SemaphoreType.DMA] * NB,
        ],
        compiler_params=pltpu.CompilerParams(use_tc_tiling_on_sc=False),
    )
    def agg_kernel(x_hbm, e_hbm, out_hbm, est, rows, zbuf, acc, gsem, ssem):
        cid = lax.axis_index("c")
        sid = lax.axis_index("s")

        pltpu.sync_copy(e_hbm.at[pl.ds(sid * BPT, BPT)],
                        est.at[pl.ds(0, BPT)])

        @pl.when(sid < XTRA)
        def _():
            pltpu.sync_copy(e_hbm.at[pl.ds(NS * BPT + sid, 1)],
                            est.at[pl.ds(BPT, 1)])

        def remap(c):
            for j in range(CHUNK // 16):
                s = pl.ds(j * 16, 16)
                est[c, 0, s] = 2 * est[c, 0, s] + cid

        def wait_gather(b):
            pltpu.make_async_copy(x_hbm.at[pl.ds(0, CHUNK)], rows.at[b], gsem[b]).wait()

        def wait_scatter(b):
            pltpu.make_async_copy(x_hbm.at[pl.ds(0, CHUNK)], rows.at[b], ssem[b]).wait()

        def gather(c, b):
            pltpu.async_copy(x_hbm.at[est.at[c, 0]], rows.at[b], gsem[b])

        def scatter(c, b):
            pltpu.async_copy(rows.at[b], acc.at[est.at[c, 1]], ssem[b], add=True)

        remap(0)
        remap(1)
        remap(2)
        gather(0, 0)
        gather(1, 1)

        def zrow(i, _):
            for j in range(DH // 16):
                zbuf[i, pl.ds(j * 16, 16)] = jnp.zeros((16,), jnp.float32)
            return 0
        lax.fori_loop(0, ZROWS, zrow, 0)
        base = sid * WB_ROWS

        @pl.when(sid < WB_TILES)
        def _():
            for k in range(WB_ROWS // ZROWS):
                pltpu.sync_copy(zbuf, acc.at[pl.ds(base + k * ZROWS, ZROWS)])
        plsc.subcore_barrier()

        def group(g, _):
            for b in range(NB):
                c = g * NB + b
                b2 = (b + 2) % NB
                c2 = c + 2

                @pl.when(c + 3 < BPT)
                def _():
                    remap(c + 3)

                @pl.when(c2 < BPT)
                def _():
                    @pl.when(c >= NB - 2)
                    def _():
                        wait_scatter(b2)
                    gather(c2, b2)

                wait_gather(b)
                scatter(c, b)
            return 0

        lax.fori_loop(0, BPT // NB, group, 0)
        for b in range(NB):
            wait_scatter(b)

        @pl.when(sid < XTRA)
        def _():
            remap(BPT)
            gather(BPT, 0)
            wait_gather(0)
            scatter(BPT, 0)
            wait_scatter(0)

        plsc.subcore_barrier()

        @pl.when(sid < WB_TILES)
        def _():
            pltpu.sync_copy(acc.at[pl.ds(base, WB_ROWS)],
                            out_hbm.at[pl.ds(base, WB_ROWS),
                                       pl.ds(cid * DH, DH)])

    return agg_kernel(x2, eidx)


def _tc_mlp_block(scale_ref, x_ref, p_ref, w1_ref, b1_ref,
                  w2_ref, b2_ref, out_ref):
    t = scale_ref[0, 0] * x_ref[...] + p_ref[...]
    h = jnp.dot(t, w1_ref[...], preferred_element_type=jnp.float32) + b1_ref[...]
    h = jnp.maximum(h, 0.0)
    out_ref[...] = (
        jnp.dot(h, w2_ref[...], preferred_element_type=jnp.float32) + b2_ref[...]
    )


def _tc_mlp(x, agg, scale, W1, b1, W2, b2):
    rows = 1000
    grid = (N // rows,)
    return pl.pallas_call(
        _tc_mlp_block,
        grid=grid,
        in_specs=[
            pl.BlockSpec(memory_space=pltpu.SMEM),
            pl.BlockSpec((rows, D), lambda i: (i, 0)),
            pl.BlockSpec((rows, D), lambda i: (i, 0)),
            pl.BlockSpec((D, D), lambda i: (0, 0)),
            pl.BlockSpec((1, D), lambda i: (0, 0)),
            pl.BlockSpec((D, D), lambda i: (0, 0)),
            pl.BlockSpec((1, D), lambda i: (0, 0)),
        ],
        out_specs=pl.BlockSpec((rows, D), lambda i: (i, 0)),
        out_shape=jax.ShapeDtypeStruct((N, D), jnp.float32),
    )(scale, x, agg, W1, b1, W2, b2)


def kernel(x, edge_index, eps, W1, b1, W2, b2):
    ei = edge_index.astype(jnp.int32)
    eidx = ei.reshape(2, NBLK, CHUNK).transpose(1, 0, 2)
    agg = _sc_aggregate(x.reshape(2 * N, DH), eidx)
    scale = (1.0 + eps).astype(jnp.float32).reshape(1, 1)
    return _tc_mlp(x, agg, scale, W1.astype(jnp.float32),
                   b1.reshape(1, D), W2.astype(jnp.float32), b2.reshape(1, D))


# speedup vs baseline: 15.6762x; 1.0254x over previous
import functools

import jax
import jax.numpy as jnp
from jax import lax
from jax.experimental import pallas as pl
from jax.experimental.pallas import tpu as pltpu
from jax.experimental.pallas import tpu_sc as plsc

N = 10000
E = 320000
D = 128
DH = D // 2

NC = 2
NS = 16

CHUNK = 128
NBLK = E // CHUNK
BPT = NBLK // NS
XTRA = NBLK - BPT * NS
NB = 4
LOOK = 2
WB_TILES = 10
WB_ROWS = N // WB_TILES
ZROWS = 200


def _sc_aggregate(x2, eidx):
    mesh = plsc.VectorSubcoreMesh(core_axis_name="c", subcore_axis_name="s")

    @functools.partial(
        pl.kernel,
        mesh=mesh,
        out_type=jax.ShapeDtypeStruct((N, D), jnp.float32),
        scratch_types=[
            pltpu.VMEM((BPT + 1, 2, CHUNK), jnp.int32),
            pltpu.VMEM((NB, CHUNK, DH), jnp.float32),
            pltpu.VMEM((ZROWS, DH), jnp.float32),
            pltpu.VMEM_SHARED((N, DH), jnp.float32),
            [pltpu.SemaphoreType.DMA] * NB,
            [pltpu.SemaphoreType.DMA] * NB,
        ],
        compiler_params=pltpu.CompilerParams(use_tc_tiling_on_sc=False),
    )
    def agg_kernel(x_hbm, e_hbm, out_hbm, est, rows, zbuf, acc, gsem, ssem):
        cid = lax.axis_index("c")
        sid = lax.axis_index("s")

        pltpu.sync_copy(e_hbm.at[pl.ds(sid * BPT, BPT)],
                        est.at[pl.ds(0, BPT)])

        @pl.when(sid < XTRA)
        def _():
            pltpu.sync_copy(e_hbm.at[pl.ds(NS * BPT + sid, 1)],
                            est.at[pl.ds(BPT, 1)])

        def remap(c):
            for j in range(CHUNK // 16):
                s = pl.ds(j * 16, 16)
                est[c, 0, s] = 2 * est[c, 0, s] + cid

        def wait_gather(b):
            pltpu.make_async_copy(x_hbm.at[pl.ds(0, CHUNK)], rows.at[b], gsem[b]).wait()

        def wait_scatter(b):
            pltpu.make_async_copy(x_hbm.at[pl.ds(0, CHUNK)], rows.at[b], ssem[b]).wait()

        def gather(c, b):
            pltpu.async_copy(x_hbm.at[est.at[c, 0]], rows.at[b], gsem[b])

        def scatter(c, b):
            pltpu.async_copy(rows.at[b], acc.at[est.at[c, 1]], ssem[b], add=True)

        for c in range(LOOK + 1):
            remap(c)
        for c in range(LOOK):
            gather(c, c)

        def zrow(i, _):
            for j in range(DH // 16):
                zbuf[i, pl.ds(j * 16, 16)] = jnp.zeros((16,), jnp.float32)
            return 0
        lax.fori_loop(0, ZROWS, zrow, 0)
        base = sid * WB_ROWS

        @pl.when(sid < WB_TILES)
        def _():
            for k in range(WB_ROWS // ZROWS):
                pltpu.sync_copy(zbuf, acc.at[pl.ds(base + k * ZROWS, ZROWS)])
        plsc.subcore_barrier()

        def group(g, _):
            for b in range(NB):
                c = g * NB + b
                b2 = (b + LOOK) % NB
                c2 = c + LOOK

                @pl.when(c + LOOK + 1 < BPT)
                def _():
                    remap(c + LOOK + 1)

                @pl.when(c2 < BPT)
                def _():
                    @pl.when(c >= NB - LOOK)
                    def _():
                        wait_scatter(b2)
                    gather(c2, b2)

                wait_gather(b)
                scatter(c, b)
            return 0

        lax.fori_loop(0, BPT // NB, group, 0)
        for b in range(NB):
            wait_scatter(b)

        @pl.when(sid < XTRA)
        def _():
            remap(BPT)
            gather(BPT, 0)
            wait_gather(0)
            scatter(BPT, 0)
            wait_scatter(0)

        plsc.subcore_barrier()

        @pl.when(sid < WB_TILES)
        def _():
            pltpu.sync_copy(acc.at[pl.ds(base, WB_ROWS)],
                            out_hbm.at[pl.ds(base, WB_ROWS),
                                       pl.ds(cid * DH, DH)])

    return agg_kernel(x2, eidx)


def _tc_mlp_block(scale_ref, x_ref, p_ref, w1_ref, b1_ref,
                  w2_ref, b2_ref, out_ref):
    t = scale_ref[0, 0] * x_ref[...] + p_ref[...]
    h = jnp.dot(t, w1_ref[...], preferred_element_type=jnp.float32) + b1_ref[...]
    h = jnp.maximum(h, 0.0)
    out_ref[...] = (
        jnp.dot(h, w2_ref[...], preferred_element_type=jnp.float32) + b2_ref[...]
    )


def _tc_mlp(x, agg, scale, W1, b1, W2, b2):
    rows = 2000
    grid = (N // rows,)
    return pl.pallas_call(
        _tc_mlp_block,
        grid=grid,
        in_specs=[
            pl.BlockSpec(memory_space=pltpu.SMEM),
            pl.BlockSpec((rows, D), lambda i: (i, 0)),
            pl.BlockSpec((rows, D), lambda i: (i, 0)),
            pl.BlockSpec((D, D), lambda i: (0, 0)),
            pl.BlockSpec((1, D), lambda i: (0, 0)),
            pl.BlockSpec((D, D), lambda i: (0, 0)),
            pl.BlockSpec((1, D), lambda i: (0, 0)),
        ],
        out_specs=pl.BlockSpec((rows, D), lambda i: (i, 0)),
        out_shape=jax.ShapeDtypeStruct((N, D), jnp.float32),
    )(scale, x, agg, W1, b1, W2, b2)


def kernel(x, edge_index, eps, W1, b1, W2, b2):
    ei = edge_index.astype(jnp.int32)
    eidx = ei.reshape(2, NBLK, CHUNK).transpose(1, 0, 2)
    agg = _sc_aggregate(x.reshape(2 * N, DH), eidx)
    scale = (1.0 + eps).astype(jnp.float32).reshape(1, 1)
    return _tc_mlp(x, agg, scale, W1.astype(jnp.float32),
                   b1.reshape(1, D), W2.astype(jnp.float32), b2.reshape(1, D))
